# baseline (speedup 1.0000x reference)
import jax
import jax.numpy as jnp
from jax import lax
from jax.experimental import pallas as pl
from jax.experimental.pallas import tpu as pltpu
from jax.experimental.pallas import tpu_sc as plsc

N = 10000
H = 8
D = 128
E = 80000
B = 100
SEG = N // B

NC = 2
NS = 16
NW = NC * NS
NPAD = 10112
RPT = NPAD // NS
EPAD = 81920
EPT = EPAD // NW
G16 = EPT // 16
CHB = 32
CH0 = 64

BN = 1000
NB = N // BN

def _proj_kernel(x_ref, w_ref, o_ref):
    o_ref[0, 0] = jnp.dot(x_ref[...], w_ref[0],
                          preferred_element_type=jnp.float32)


def _run_proj(x, Wstk):
    return pl.pallas_call(
        _proj_kernel,
        grid=(4, H, NB),
        in_specs=[
            pl.BlockSpec((BN, D), lambda c, h, nb: (nb, 0)),
            pl.BlockSpec((1, D, D), lambda c, h, nb: (c, 0, h)),
        ],
        out_specs=pl.BlockSpec((1, 1, BN, D), lambda c, h, nb: (c, h, nb, 0)),
        out_shape=jax.ShapeDtypeStruct((4, H, N, D), jnp.float32),
    )(x, Wstk)


def _logit_kernel(x_ref, w_ref, al_ref, ar_ref, elp_ref, erp_ref, m_ref,
                  mel_ref, mer_ref):
    nb = pl.program_id(1)
    x = x_ref[...]
    zpad = jnp.zeros((D, 120), jnp.float32)
    vl, vr = [], []
    for h in range(H):
        wh = w_ref[0, :, h * D:(h + 1) * D]
        vl.append(jnp.dot(wh, al_ref[0, h, :][:, None],
                          preferred_element_type=jnp.float32))
        vr.append(jnp.dot(wh, ar_ref[0, h, :][:, None],
                          preferred_element_type=jnp.float32))
    Vl = jnp.concatenate(vl + [zpad], axis=1)
    Vr = jnp.concatenate(vr + [zpad], axis=1)
    el = jnp.dot(x, Vl, preferred_element_type=jnp.float32)
    er = jnp.dot(x, Vr, preferred_element_type=jnp.float32)
    elp_ref[0] = el
    erp_ref[0] = er
    me = jnp.broadcast_to(jnp.max(el[:, 0:16], axis=0, keepdims=True), (8, 16))
    mr = jnp.broadcast_to(jnp.max(er[:, 0:16], axis=0, keepdims=True), (8, 16))

    @pl.when(nb == 0)
    def _():
        mel_ref[...] = me
        mer_ref[...] = mr

    @pl.when(nb > 0)
    def _():
        mel_ref[...] = jnp.maximum(mel_ref[...], me)
        mer_ref[...] = jnp.maximum(mer_ref[...], mr)

    @pl.when(nb == NB - 1)
    def _():
        m_ref[0] = jnp.maximum(mel_ref[...] + mer_ref[...], 0.0)


def _run_logits(x, Wstk, alstk, arstk):
    return pl.pallas_call(
        _logit_kernel,
        grid=(4, NB),
        in_specs=[
            pl.BlockSpec((BN, D), lambda c, nb: (nb, 0)),
            pl.BlockSpec((1, D, H * D), lambda c, nb: (c, 0, 0)),
            pl.BlockSpec((1, H, D), lambda c, nb: (c, 0, 0)),
            pl.BlockSpec((1, H, D), lambda c, nb: (c, 0, 0)),
        ],
        out_specs=[
            pl.BlockSpec((1, BN, 128), lambda c, nb: (c, nb, 0)),
            pl.BlockSpec((1, BN, 128), lambda c, nb: (c, nb, 0)),
            pl.BlockSpec((1, 8, 16), lambda c, nb: (c, 0, 0)),
        ],
        out_shape=[
            jax.ShapeDtypeStruct((4, NPAD, 128), jnp.float32),
            jax.ShapeDtypeStruct((4, NPAD, 128), jnp.float32),
            jax.ShapeDtypeStruct((4, 8, 16), jnp.float32),
        ],
        scratch_shapes=[
            pltpu.VMEM((8, 16), jnp.float32),
            pltpu.VMEM((8, 16), jnp.float32),
        ],
    )(x, Wstk, alstk, arstk)


NCK = EPT // CHB


def _att_kernel(elp_h, erp_h, m_h, src_h, dst_h, zden_h,
                w_out_h, dpart_h,
                src_v, dst_v, elr_v, err_v, wb_v, wpad_v, m_v, den_s,
                sem0, sem1):
    cid = lax.axis_index("c")
    sid = lax.axis_index("s")
    wid = sid * NC + cid
    pltpu.sync_copy(m_h, m_v)
    r0 = sid * RPT
    z16 = jnp.zeros((16,), jnp.float32)

    @pl.loop(0, CHB)
    def _zr(i):
        for k in range(8):
            wpad_v.at[i][pl.ds(k * 16, 16)] = z16

    for c in range(4):
        pltpu.sync_copy(zden_h, den_s.at[pl.ds(r0, RPT)])
        pltpu.sync_copy(src_h.at[c, pl.ds(wid * EPT, EPT)], src_v)
        pltpu.sync_copy(dst_h.at[c, pl.ds(wid * EPT, EPT)], dst_v)
        plsc.subcore_barrier()
        m16 = m_v.at[c, 0][...]
        sems = (sem0, sem1)
        bufs = ((elr_v.at[0], err_v.at[0]), (elr_v.at[1], err_v.at[1]))

        def _start(k, p):
            pltpu.async_copy(
                elp_h.at[c].at[src_v.at[pl.ds(k * CHB, CHB)]],
                bufs[p][0], sems[p])
            pltpu.async_copy(
                erp_h.at[c].at[dst_v.at[pl.ds(k * CHB, CHB)]],
                bufs[p][1], sems[p])

        def _finish(k, p):
            pltpu.make_async_copy(elp_h.at[c].at[
                src_v.at[pl.ds(k * CHB, CHB)]], bufs[p][0], sems[p]).wait()
            pltpu.make_async_copy(erp_h.at[c].at[
                dst_v.at[pl.ds(k * CHB, CHB)]], bufs[p][1], sems[p]).wait()

            @pl.loop(0, CHB)
            def _row(i):
                e = (bufs[p][0].at[i][pl.ds(0, 16)]
                     + bufs[p][1].at[i][pl.ds(0, 16)])
                s = jnp.maximum(e, 0.2 * e)
                w16 = jnp.exp(s - m16)
                wb_v.at[i][...] = w16
                wpad_v.at[i][pl.ds(0, 16)] = w16

            pltpu.sync_copy(
                wb_v, w_out_h.at[c, pl.ds(wid * EPT + k * CHB, CHB)])
            pltpu.sync_copy(
                wpad_v, den_s.at[dst_v.at[pl.ds(k * CHB, CHB)]], add=True)

        _start(0, 0)

        @pl.loop(0, NCK, step=2)
        def _chunk(k):
            _start(k + 1, 1)
            _finish(k, 0)

            @pl.when(k + 2 < NCK)
            def _():
                _start(k + 2, 0)

            _finish(k + 1, 1)

        plsc.subcore_barrier()
        pltpu.sync_copy(den_s.at[pl.ds(r0, RPT)],
                        dpart_h.at[c, cid, pl.ds(r0, RPT)])
        plsc.subcore_barrier()


def _run_att(elp, erp, mshift, srcp, dstp, zden):
    mesh = plsc.VectorSubcoreMesh(core_axis_name="c", subcore_axis_name="s")
    return pl.kernel(
        _att_kernel,
        out_type=(
            jax.ShapeDtypeStruct((4, EPAD, 16), jnp.float32),
            jax.ShapeDtypeStruct((4, NC, NPAD, 128), jnp.float32),
        ),
        mesh=mesh,
        scratch_types=[
            pltpu.VMEM((EPT,), jnp.int32),
            pltpu.VMEM((EPT,), jnp.int32),
            pltpu.VMEM((2, CHB, 128), jnp.float32),
            pltpu.VMEM((2, CHB, 128), jnp.float32),
            pltpu.VMEM((CHB, 16), jnp.float32),
            pltpu.VMEM((CHB, 128), jnp.float32),
            pltpu.VMEM((4, 8, 16), jnp.float32),
            pltpu.VMEM_SHARED((NPAD, 128), jnp.float32),
            pltpu.SemaphoreType.DMA,
            pltpu.SemaphoreType.DMA,
        ],
        compiler_params=pltpu.CompilerParams(needs_layout_passes=False),
    )(elp, erp, mshift, srcp, dstp, zden)


def _rden_kernel(dp_ref, o_ref):
    d = dp_ref[0, 0] + dp_ref[0, 1]
    o_ref[0] = 1.0 / d


def _run_rden(dpstk):
    return pl.pallas_call(
        _rden_kernel,
        grid=(4, NS),
        in_specs=[pl.BlockSpec((1, NC, RPT, 128), lambda c, i: (c, 0, i, 0))],
        out_specs=pl.BlockSpec((1, RPT, 128), lambda c, i: (c, i, 0)),
        out_shape=jax.ShapeDtypeStruct((4, NPAD, 128), jnp.float32),
    )(dpstk)


GE = 128
NGE = EPT // GE
GED = 32
NGED = EPT // GED


def _alpha_kernel(w_h, rden_h, dst2_h, alphaT_h,
                  dst2_v, wtmp_v, rows_v, at_v, sem0, sem1):
    cid = lax.axis_index("c")
    sid = lax.axis_index("s")
    wid = sid * NC + cid
    iota16 = lax.iota(jnp.int32, 16)

    for c in range(4):
        pltpu.sync_copy(dst2_h.at[c, wid], dst2_v)
        sems = (sem0, sem1)

        def _start(s, p):
            pltpu.async_copy(rden_h.at[c].at[dst2_v.at[s]],
                             rows_v.at[p], sems[p])

        def _finish(s, p):
            sb = s * GE
            pltpu.sync_copy(
                w_h.at[c, pl.ds((wid * NGE + s) * GE, GE)], wtmp_v)
            pltpu.make_async_copy(rden_h.at[c].at[dst2_v.at[s]],
                                  rows_v.at[p], sems[p]).wait()

            @pl.loop(0, GE)
            def _row(i):
                wtmp_v.at[i][...] = (
                    wtmp_v.at[i][...] * rows_v.at[p, i][pl.ds(0, 16)])

            for g in range(GE // 16):
                ridx = g * 16 + iota16
                for h in range(H):
                    vec = plsc.load_gather(
                        wtmp_v, [ridx, jnp.full((16,), h, jnp.int32)])
                    at_v[pl.ds(h * EPT + sb + g * 16, 16)] = vec

        _start(0, 0)

        @pl.loop(0, NGE, step=2)
        def _sub(s):
            _start(s + 1, 1)
            _finish(s, 0)

            @pl.when(s + 2 < NGE)
            def _():
                _start(s + 2, 0)

            _finish(s + 1, 1)

        for h in range(H):
            pltpu.sync_copy(
                at_v.at[pl.ds(h * EPT, EPT)],
                alphaT_h.at[pl.ds(((c * H + h) * NW + wid) * EPT, EPT)])


def _run_alpha(wstk, rden, dst2):
    mesh = plsc.VectorSubcoreMesh(core_axis_name="c", subcore_axis_name="s")
    return pl.kernel(
        _alpha_kernel,
        out_type=jax.ShapeDtypeStruct((4 * H * EPAD,), jnp.float32),
        mesh=mesh,
        scratch_types=[
            pltpu.VMEM((NGE, GE), jnp.int32),
            pltpu.VMEM((GE, 16), jnp.float32),
            pltpu.VMEM((2, GE, D), jnp.float32),
            pltpu.VMEM((H * EPT,), jnp.float32),
            pltpu.SemaphoreType.DMA,
            pltpu.SemaphoreType.DMA,
        ],
        compiler_params=pltpu.CompilerParams(needs_layout_passes=False),
    )(wstk, rden, dst2)


def _agg_kernel(hT_h, alphaT_h, src2_h, dst2_h,
                part_h,
                src2_v, dst2_v, ac_v, rows_v, zv_v, acc_s,
                sem0, sem1, sem2, sem3, ssem0, ssem1, ssem2, ssem3):
    cid = lax.axis_index("c")
    sid = lax.axis_index("s")
    wid = sid * NC + cid
    z16 = jnp.zeros((16,), jnp.float32)

    @pl.loop(0, RPT // 8)
    def _zr(i):
        for k in range(8):
            zv_v.at[i][pl.ds(k * 16, 16)] = z16

    @pl.loop(0, H)
    def _hloop(h):
        for q in range(8):
            pltpu.sync_copy(
                zv_v, acc_s.at[pl.ds(sid * RPT + q * (RPT // 8), RPT // 8)])
        plsc.subcore_barrier()
        for c in range(4):
            pltpu.sync_copy(src2_h.at[c, wid], src2_v)
            pltpu.sync_copy(dst2_h.at[c, wid], dst2_v)
            pltpu.sync_copy(
                alphaT_h.at[pl.ds(((c * H * NW) * EPT
                                   + (h * NW + wid) * EPT), EPT)], ac_v)
            sems = (sem0, sem1, sem2, sem3)
            ssems = (ssem0, ssem1, ssem2, ssem3)

            def _wait_scat(p):
                pltpu.make_async_copy(rows_v.at[p],
                                      acc_s.at[dst2_v.at[0]],
                                      ssems[p]).wait()

            def _start(g, p):
                pltpu.async_copy(hT_h.at[c, h].at[src2_v.at[g]],
                                 rows_v.at[p], sems[p])

            def _finish(g, p):
                pltpu.make_async_copy(hT_h.at[c, h].at[src2_v.at[g]],
                                      rows_v.at[p], sems[p]).wait()

                @pl.loop(0, GED // 16)
                def _sg(jj):
                    av = ac_v[pl.ds(g * GED + jj * 16, 16)]
                    for j in range(16):
                        a = av[j]
                        rv = rows_v.at[p, jj * 16 + j]
                        for k in range(D // 16):
                            rv[pl.ds(k * 16, 16)] = rv[pl.ds(k * 16, 16)] * a

                pltpu.async_copy(rows_v.at[p], acc_s.at[dst2_v.at[g]],
                                 ssems[p], add=True)

            _start(0, 0)
            _start(1, 1)

            @pl.loop(0, NGED, step=4)
            def _grp(g):
                for q in range(4):
                    n = g + q
                    p2 = (q + 2) % 4

                    @pl.when(n + 2 < NGED)
                    def _():
                        @pl.when(n >= 2)
                        def _():
                            _wait_scat(p2)

                        _start(n + 2, p2)

                    _finish(n, q)

            for p in range(4):
                _wait_scat(p)

        plsc.subcore_barrier()
        pltpu.sync_copy(acc_s.at[pl.ds(sid * RPT, RPT)],
                        part_h.at[cid, h, pl.ds(sid * RPT, RPT)])


def _run_agg(hT, alphaT, src2d, dst2d):
    mesh = plsc.VectorSubcoreMesh(core_axis_name="c", subcore_axis_name="s")
    return pl.kernel(
        _agg_kernel,
        out_type=jax.ShapeDtypeStruct((NC, H, NPAD, D), jnp.float32),
        mesh=mesh,
        scratch_types=[
            pltpu.VMEM((NGED, GED), jnp.int32),
            pltpu.VMEM((NGED, GED), jnp.int32),
            pltpu.VMEM((EPT,), jnp.float32),
            pltpu.VMEM((4, GED, D), jnp.float32),
            pltpu.VMEM((RPT // 8, D), jnp.float32),
            pltpu.VMEM_SHARED((NPAD, D), jnp.float32),
            pltpu.SemaphoreType.DMA,
            pltpu.SemaphoreType.DMA,
            pltpu.SemaphoreType.DMA,
            pltpu.SemaphoreType.DMA,
            pltpu.SemaphoreType.DMA,
            pltpu.SemaphoreType.DMA,
            pltpu.SemaphoreType.DMA,
            pltpu.SemaphoreType.DMA,
        ],
        compiler_params=pltpu.CompilerParams(needs_layout_passes=False),
    )(hT, alphaT, src2d, dst2d)


SEGB = 4
ROWS = SEGB * SEG


def _combine_kernel(x_ref, p_ref, rw_ref, b_ref, o_ref):
    x = x_ref[...]
    rws = jnp.sum(rw_ref[...], axis=0)
    bs = jnp.sum(b_ref[...], axis=0)
    res = jnp.dot(x, rws, preferred_element_type=jnp.float32) + bs[None, :]
    m = p_ref[0, 0] + p_ref[1, 0] + res[:, 0:D]
    for h in range(1, H):
        v = p_ref[0, h] + p_ref[1, h] + res[:, h * D:(h + 1) * D]
        m = jnp.maximum(m, v)
    xb = x.reshape(SEGB, SEG, D)
    mean = jnp.mean(xb, axis=1)
    mean = jnp.broadcast_to(mean[:, None, :], (SEGB, SEG, D)).reshape(ROWS, D)
    o_ref[...] = m + mean


def _run_combine(x, part, rWstk, bstk):
    return pl.pallas_call(
        _combine_kernel,
        grid=(B // SEGB,),
        in_specs=[
            pl.BlockSpec((ROWS, D), lambda i: (i, 0)),
            pl.BlockSpec((NC, H, ROWS, D), lambda i: (0, 0, i, 0)),
            pl.BlockSpec((4, D, H * D), lambda i: (0, 0, 0)),
            pl.BlockSpec((4, H * D), lambda i: (0, 0)),
        ],
        out_specs=pl.BlockSpec((ROWS, D), lambda i: (i, 0)),
        out_shape=jax.ShapeDtypeStruct((N, D), jnp.float32),
    )(x, part, rWstk, bstk)


def kernel(x, edge_intra, edge_inter, batch_num_nodes,
           W_i1f, al_i1f, ar_i1f, rW_i1f, b_i1f,
           W_inf, al_inf, ar_inf, rW_inf, b_inf,
           W_i1b, al_i1b, ar_i1b, rW_i1b, b_i1b,
           W_inb, al_inb, ar_inb, rW_inb, b_inb):
    si, di = edge_intra[0], edge_intra[1]
    se, de = edge_inter[0], edge_inter[1]

    Wstk = jnp.stack([W_i1f, W_inf, W_i1b, W_inb])
    alstk = jnp.stack([al_i1f, al_inf, al_i1b, al_inb])
    arstk = jnp.stack([ar_i1f, ar_inf, ar_i1b, ar_inb])
    rWstk = jnp.stack([rW_i1f, rW_inf, rW_i1b, rW_inb])
    bstk = jnp.stack([b_i1f, b_inf, b_i1b, b_inb])

    srcs = jnp.stack([si, se, di, de])
    dsts = jnp.stack([di, de, si, se])
    ppt = (EPAD - E) // NW
    rpt_e = E // NW
    pad_dst = N + (jnp.arange(NW * ppt, dtype=jnp.int32)
                   % (NPAD - N)).reshape(1, NW, ppt)
    pad_dst = jnp.broadcast_to(pad_dst, (4, NW, ppt))
    srcp = jnp.concatenate(
        [srcs.reshape(4, NW, rpt_e),
         jnp.zeros((4, NW, ppt), jnp.int32)], axis=2).reshape(4, EPAD)
    dstp = jnp.concatenate(
        [dsts.reshape(4, NW, rpt_e), pad_dst], axis=2).reshape(4, EPAD)
    dst2 = dstp.reshape(4, NW, NGE, GE)
    src2d = srcp.reshape(4, NW, NGED, GED)
    dst2d = dstp.reshape(4, NW, NGED, GED)

    zden = jnp.zeros((RPT, 128), jnp.float32)
    zacc = jnp.zeros((RPT, D), jnp.float32)

    hT = _run_proj(x, Wstk)
    elp, erp, mshift = _run_logits(x, Wstk, alstk, arstk)

    wstk, dpstk = _run_att(elp, erp, mshift, srcp, dstp, zden)

    rden = _run_rden(dpstk)
    alphaT = _run_alpha(wstk, rden, dst2)
    part = _run_agg(hT, alphaT, src2d, dst2d)

    return _run_combine(x, part, rWstk, bstk)


# speedup vs baseline: 1.0250x; 1.0250x over previous
import jax
import jax.numpy as jnp
from jax import lax
from jax.experimental import pallas as pl
from jax.experimental.pallas import tpu as pltpu
from jax.experimental.pallas import tpu_sc as plsc

N = 10000
H = 8
D = 128
E = 80000
B = 100
SEG = N // B

NC = 2
NS = 16
NW = NC * NS
NPAD = 10112
RPT = NPAD // NS
EPAD = 81920
EPT = EPAD // NW
G16 = EPT // 16
CHB = 32
CH0 = 64

BN = 1000
NB = N // BN

def _proj_kernel(x_ref, w_ref, o_ref):
    o_ref[0, 0] = jnp.dot(x_ref[...], w_ref[0],
                          preferred_element_type=jnp.float32)


def _run_proj(x, Wstk):
    return pl.pallas_call(
        _proj_kernel,
        grid=(4, H, NB),
        in_specs=[
            pl.BlockSpec((BN, D), lambda c, h, nb: (nb, 0)),
            pl.BlockSpec((1, D, D), lambda c, h, nb: (c, 0, h)),
        ],
        out_specs=pl.BlockSpec((1, 1, BN, D), lambda c, h, nb: (c, h, nb, 0)),
        out_shape=jax.ShapeDtypeStruct((4, H, N, D), jnp.float32),
    )(x, Wstk)


def _logit_kernel(x_ref, w_ref, al_ref, ar_ref, elp_ref, erp_ref, m_ref,
                  mel_ref, mer_ref):
    nb = pl.program_id(1)
    x = x_ref[...]
    zpad = jnp.zeros((D, 120), jnp.float32)
    vl, vr = [], []
    for h in range(H):
        wh = w_ref[0, :, h * D:(h + 1) * D]
        vl.append(jnp.dot(wh, al_ref[0, h, :][:, None],
                          preferred_element_type=jnp.float32))
        vr.append(jnp.dot(wh, ar_ref[0, h, :][:, None],
                          preferred_element_type=jnp.float32))
    Vl = jnp.concatenate(vl + [zpad], axis=1)
    Vr = jnp.concatenate(vr + [zpad], axis=1)
    el = jnp.dot(x, Vl, preferred_element_type=jnp.float32)
    er = jnp.dot(x, Vr, preferred_element_type=jnp.float32)
    elp_ref[0] = el
    erp_ref[0] = er
    me = jnp.broadcast_to(jnp.max(el[:, 0:16], axis=0, keepdims=True), (8, 16))
    mr = jnp.broadcast_to(jnp.max(er[:, 0:16], axis=0, keepdims=True), (8, 16))

    @pl.when(nb == 0)
    def _():
        mel_ref[...] = me
        mer_ref[...] = mr

    @pl.when(nb > 0)
    def _():
        mel_ref[...] = jnp.maximum(mel_ref[...], me)
        mer_ref[...] = jnp.maximum(mer_ref[...], mr)

    @pl.when(nb == NB - 1)
    def _():
        m_ref[0] = jnp.maximum(mel_ref[...] + mer_ref[...], 0.0)


def _run_logits(x, Wstk, alstk, arstk):
    return pl.pallas_call(
        _logit_kernel,
        grid=(4, NB),
        in_specs=[
            pl.BlockSpec((BN, D), lambda c, nb: (nb, 0)),
            pl.BlockSpec((1, D, H * D), lambda c, nb: (c, 0, 0)),
            pl.BlockSpec((1, H, D), lambda c, nb: (c, 0, 0)),
            pl.BlockSpec((1, H, D), lambda c, nb: (c, 0, 0)),
        ],
        out_specs=[
            pl.BlockSpec((1, BN, 128), lambda c, nb: (c, nb, 0)),
            pl.BlockSpec((1, BN, 128), lambda c, nb: (c, nb, 0)),
            pl.BlockSpec((1, 8, 16), lambda c, nb: (c, 0, 0)),
        ],
        out_shape=[
            jax.ShapeDtypeStruct((4, NPAD, 128), jnp.float32),
            jax.ShapeDtypeStruct((4, NPAD, 128), jnp.float32),
            jax.ShapeDtypeStruct((4, 8, 16), jnp.float32),
        ],
        scratch_shapes=[
            pltpu.VMEM((8, 16), jnp.float32),
            pltpu.VMEM((8, 16), jnp.float32),
        ],
    )(x, Wstk, alstk, arstk)


NCK = EPT // CHB


def _att_kernel(elp_h, erp_h, m_h, src_h, dst_h, zden_h,
                w_out_h, dpart_h,
                src_v, dst_v, elr_v, err_v, wb_v, wpad_v, m_v, den_s,
                sem0, sem1):
    cid = lax.axis_index("c")
    sid = lax.axis_index("s")
    wid = sid * NC + cid
    pltpu.sync_copy(m_h, m_v)
    r0 = sid * RPT
    z16 = jnp.zeros((16,), jnp.float32)

    @pl.loop(0, CHB)
    def _zr(i):
        for k in range(8):
            wpad_v.at[i][pl.ds(k * 16, 16)] = z16

    for c in range(4):
        pltpu.sync_copy(zden_h, den_s.at[pl.ds(r0, RPT)])
        pltpu.sync_copy(src_h.at[c, pl.ds(wid * EPT, EPT)], src_v)
        pltpu.sync_copy(dst_h.at[c, pl.ds(wid * EPT, EPT)], dst_v)
        plsc.subcore_barrier()
        m16 = m_v.at[c, 0][...]
        sems = (sem0, sem1)
        bufs = ((elr_v.at[0], err_v.at[0]), (elr_v.at[1], err_v.at[1]))

        def _start(k, p):
            pltpu.async_copy(
                elp_h.at[c].at[src_v.at[pl.ds(k * CHB, CHB)]],
                bufs[p][0], sems[p])
            pltpu.async_copy(
                erp_h.at[c].at[dst_v.at[pl.ds(k * CHB, CHB)]],
                bufs[p][1], sems[p])

        def _finish(k, p):
            pltpu.make_async_copy(elp_h.at[c].at[
                src_v.at[pl.ds(k * CHB, CHB)]], bufs[p][0], sems[p]).wait()
            pltpu.make_async_copy(erp_h.at[c].at[
                dst_v.at[pl.ds(k * CHB, CHB)]], bufs[p][1], sems[p]).wait()

            @pl.loop(0, CHB)
            def _row(i):
                e = (bufs[p][0].at[i][pl.ds(0, 16)]
                     + bufs[p][1].at[i][pl.ds(0, 16)])
                s = jnp.maximum(e, 0.2 * e)
                w16 = jnp.exp(s - m16)
                wb_v.at[i][...] = w16
                wpad_v.at[i][pl.ds(0, 16)] = w16

            pltpu.sync_copy(
                wb_v, w_out_h.at[c, pl.ds(wid * EPT + k * CHB, CHB)])
            pltpu.sync_copy(
                wpad_v, den_s.at[dst_v.at[pl.ds(k * CHB, CHB)]], add=True)

        _start(0, 0)

        @pl.loop(0, NCK, step=2)
        def _chunk(k):
            _start(k + 1, 1)
            _finish(k, 0)

            @pl.when(k + 2 < NCK)
            def _():
                _start(k + 2, 0)

            _finish(k + 1, 1)

        plsc.subcore_barrier()
        pltpu.sync_copy(den_s.at[pl.ds(r0, RPT)],
                        dpart_h.at[c, cid, pl.ds(r0, RPT)])
        plsc.subcore_barrier()


def _run_att(elp, erp, mshift, srcp, dstp, zden):
    mesh = plsc.VectorSubcoreMesh(core_axis_name="c", subcore_axis_name="s")
    return pl.kernel(
        _att_kernel,
        out_type=(
            jax.ShapeDtypeStruct((4, EPAD, 16), jnp.float32),
            jax.ShapeDtypeStruct((4, NC, NPAD, 128), jnp.float32),
        ),
        mesh=mesh,
        scratch_types=[
            pltpu.VMEM((EPT,), jnp.int32),
            pltpu.VMEM((EPT,), jnp.int32),
            pltpu.VMEM((2, CHB, 128), jnp.float32),
            pltpu.VMEM((2, CHB, 128), jnp.float32),
            pltpu.VMEM((CHB, 16), jnp.float32),
            pltpu.VMEM((CHB, 128), jnp.float32),
            pltpu.VMEM((4, 8, 16), jnp.float32),
            pltpu.VMEM_SHARED((NPAD, 128), jnp.float32),
            pltpu.SemaphoreType.DMA,
            pltpu.SemaphoreType.DMA,
        ],
        compiler_params=pltpu.CompilerParams(needs_layout_passes=False),
    )(elp, erp, mshift, srcp, dstp, zden)


def _rden_kernel(dp_ref, o_ref):
    d = dp_ref[0, 0] + dp_ref[0, 1]
    o_ref[0] = 1.0 / d


def _run_rden(dpstk):
    return pl.pallas_call(
        _rden_kernel,
        grid=(4, NS),
        in_specs=[pl.BlockSpec((1, NC, RPT, 128), lambda c, i: (c, 0, i, 0))],
        out_specs=pl.BlockSpec((1, RPT, 128), lambda c, i: (c, i, 0)),
        out_shape=jax.ShapeDtypeStruct((4, NPAD, 128), jnp.float32),
    )(dpstk)


GE = 128
NGE = EPT // GE
GED = 64
NGED = EPT // GED


def _alpha_kernel(w_h, rden_h, dst2_h, alphaT_h,
                  dst2_v, wtmp_v, rows_v, at_v, sem0, sem1):
    cid = lax.axis_index("c")
    sid = lax.axis_index("s")
    wid = sid * NC + cid
    iota16 = lax.iota(jnp.int32, 16)

    for c in range(4):
        pltpu.sync_copy(dst2_h.at[c, wid], dst2_v)
        sems = (sem0, sem1)

        def _start(s, p):
            pltpu.async_copy(rden_h.at[c].at[dst2_v.at[s]],
                             rows_v.at[p], sems[p])

        def _finish(s, p):
            sb = s * GE
            pltpu.sync_copy(
                w_h.at[c, pl.ds((wid * NGE + s) * GE, GE)], wtmp_v)
            pltpu.make_async_copy(rden_h.at[c].at[dst2_v.at[s]],
                                  rows_v.at[p], sems[p]).wait()

            @pl.loop(0, GE)
            def _row(i):
                wtmp_v.at[i][...] = (
                    wtmp_v.at[i][...] * rows_v.at[p, i][pl.ds(0, 16)])

            for g in range(GE // 16):
                ridx = g * 16 + iota16
                for h in range(H):
                    vec = plsc.load_gather(
                        wtmp_v, [ridx, jnp.full((16,), h, jnp.int32)])
                    at_v[pl.ds(h * EPT + sb + g * 16, 16)] = vec

        _start(0, 0)

        @pl.loop(0, NGE, step=2)
        def _sub(s):
            _start(s + 1, 1)
            _finish(s, 0)

            @pl.when(s + 2 < NGE)
            def _():
                _start(s + 2, 0)

            _finish(s + 1, 1)

        for h in range(H):
            pltpu.sync_copy(
                at_v.at[pl.ds(h * EPT, EPT)],
                alphaT_h.at[pl.ds(((c * H + h) * NW + wid) * EPT, EPT)])


def _run_alpha(wstk, rden, dst2):
    mesh = plsc.VectorSubcoreMesh(core_axis_name="c", subcore_axis_name="s")
    return pl.kernel(
        _alpha_kernel,
        out_type=jax.ShapeDtypeStruct((4 * H * EPAD,), jnp.float32),
        mesh=mesh,
        scratch_types=[
            pltpu.VMEM((NGE, GE), jnp.int32),
            pltpu.VMEM((GE, 16), jnp.float32),
            pltpu.VMEM((2, GE, D), jnp.float32),
            pltpu.VMEM((H * EPT,), jnp.float32),
            pltpu.SemaphoreType.DMA,
            pltpu.SemaphoreType.DMA,
        ],
        compiler_params=pltpu.CompilerParams(needs_layout_passes=False),
    )(wstk, rden, dst2)


def _agg_kernel(hT_h, alphaT_h, src2_h, dst2_h,
                part_h,
                src2_v, dst2_v, ac_v, rows_v, zv_v, acc_s,
                sem0, sem1, sem2, ssem0, ssem1, ssem2):
    cid = lax.axis_index("c")
    sid = lax.axis_index("s")
    wid = sid * NC + cid
    z16 = jnp.zeros((16,), jnp.float32)

    @pl.loop(0, RPT // 8)
    def _zr(i):
        for k in range(8):
            zv_v.at[i][pl.ds(k * 16, 16)] = z16

    @pl.loop(0, H)
    def _hloop(h):
        for q in range(8):
            pltpu.sync_copy(
                zv_v, acc_s.at[pl.ds(sid * RPT + q * (RPT // 8), RPT // 8)])
        plsc.subcore_barrier()
        for c in range(4):
            pltpu.sync_copy(src2_h.at[c, wid], src2_v)
            pltpu.sync_copy(dst2_h.at[c, wid], dst2_v)
            pltpu.sync_copy(
                alphaT_h.at[pl.ds(((c * H * NW) * EPT
                                   + (h * NW + wid) * EPT), EPT)], ac_v)
            sems = (sem0, sem1, sem2)
            ssems = (ssem0, ssem1, ssem2)

            def _wait_scat(p):
                pltpu.make_async_copy(rows_v.at[p],
                                      acc_s.at[dst2_v.at[0]],
                                      ssems[p]).wait()

            def _start(g, p):
                pltpu.async_copy(hT_h.at[c, h].at[src2_v.at[g]],
                                 rows_v.at[p], sems[p])

            def _finish(g, p):
                pltpu.make_async_copy(hT_h.at[c, h].at[src2_v.at[g]],
                                      rows_v.at[p], sems[p]).wait()

                @pl.loop(0, GED // 16)
                def _sg(jj):
                    av = ac_v[pl.ds(g * GED + jj * 16, 16)]
                    for j in range(16):
                        a = av[j]
                        rv = rows_v.at[p, jj * 16 + j]
                        for k in range(D // 16):
                            rv[pl.ds(k * 16, 16)] = rv[pl.ds(k * 16, 16)] * a

                pltpu.async_copy(rows_v.at[p], acc_s.at[dst2_v.at[g]],
                                 ssems[p], add=True)

            _start(0, 0)

            @pl.loop(0, NGED - 1, step=3)
            def _grp(g):
                for q in range(3):
                    n = g + q
                    p1 = (q + 1) % 3

                    @pl.when(n >= 2)
                    def _():
                        _wait_scat(p1)

                    _start(n + 1, p1)
                    _finish(n, q)

            _finish(NGED - 1, (NGED - 1) % 3)
            for p in range(3):
                _wait_scat(p)

        plsc.subcore_barrier()
        pltpu.sync_copy(acc_s.at[pl.ds(sid * RPT, RPT)],
                        part_h.at[cid, h, pl.ds(sid * RPT, RPT)])


def _run_agg(hT, alphaT, src2d, dst2d):
    mesh = plsc.VectorSubcoreMesh(core_axis_name="c", subcore_axis_name="s")
    return pl.kernel(
        _agg_kernel,
        out_type=jax.ShapeDtypeStruct((NC, H, NPAD, D), jnp.float32),
        mesh=mesh,
        scratch_types=[
            pltpu.VMEM((NGED, GED), jnp.int32),
            pltpu.VMEM((NGED, GED), jnp.int32),
            pltpu.VMEM((EPT,), jnp.float32),
            pltpu.VMEM((3, GED, D), jnp.float32),
            pltpu.VMEM((RPT // 8, D), jnp.float32),
            pltpu.VMEM_SHARED((NPAD, D), jnp.float32),
            pltpu.SemaphoreType.DMA,
            pltpu.SemaphoreType.DMA,
            pltpu.SemaphoreType.DMA,
            pltpu.SemaphoreType.DMA,
            pltpu.SemaphoreType.DMA,
            pltpu.SemaphoreType.DMA,
        ],
        compiler_params=pltpu.CompilerParams(needs_layout_passes=False),
    )(hT, alphaT, src2d, dst2d)


SEGB = 4
ROWS = SEGB * SEG


def _combine_kernel(x_ref, p_ref, rw_ref, b_ref, o_ref):
    x = x_ref[...]
    rws = jnp.sum(rw_ref[...], axis=0)
    bs = jnp.sum(b_ref[...], axis=0)
    res = jnp.dot(x, rws, preferred_element_type=jnp.float32) + bs[None, :]
    m = p_ref[0, 0] + p_ref[1, 0] + res[:, 0:D]
    for h in range(1, H):
        v = p_ref[0, h] + p_ref[1, h] + res[:, h * D:(h + 1) * D]
        m = jnp.maximum(m, v)
    xb = x.reshape(SEGB, SEG, D)
    mean = jnp.mean(xb, axis=1)
    mean = jnp.broadcast_to(mean[:, None, :], (SEGB, SEG, D)).reshape(ROWS, D)
    o_ref[...] = m + mean


def _run_combine(x, part, rWstk, bstk):
    return pl.pallas_call(
        _combine_kernel,
        grid=(B // SEGB,),
        in_specs=[
            pl.BlockSpec((ROWS, D), lambda i: (i, 0)),
            pl.BlockSpec((NC, H, ROWS, D), lambda i: (0, 0, i, 0)),
            pl.BlockSpec((4, D, H * D), lambda i: (0, 0, 0)),
            pl.BlockSpec((4, H * D), lambda i: (0, 0)),
        ],
        out_specs=pl.BlockSpec((ROWS, D), lambda i: (i, 0)),
        out_shape=jax.ShapeDtypeStruct((N, D), jnp.float32),
    )(x, part, rWstk, bstk)


def kernel(x, edge_intra, edge_inter, batch_num_nodes,
           W_i1f, al_i1f, ar_i1f, rW_i1f, b_i1f,
           W_inf, al_inf, ar_inf, rW_inf, b_inf,
           W_i1b, al_i1b, ar_i1b, rW_i1b, b_i1b,
           W_inb, al_inb, ar_inb, rW_inb, b_inb):
    si, di = edge_intra[0], edge_intra[1]
    se, de = edge_inter[0], edge_inter[1]

    Wstk = jnp.stack([W_i1f, W_inf, W_i1b, W_inb])
    alstk = jnp.stack([al_i1f, al_inf, al_i1b, al_inb])
    arstk = jnp.stack([ar_i1f, ar_inf, ar_i1b, ar_inb])
    rWstk = jnp.stack([rW_i1f, rW_inf, rW_i1b, rW_inb])
    bstk = jnp.stack([b_i1f, b_inf, b_i1b, b_inb])

    srcs = jnp.stack([si, se, di, de])
    dsts = jnp.stack([di, de, si, se])
    ppt = (EPAD - E) // NW
    rpt_e = E // NW
    pad_dst = N + (jnp.arange(NW * ppt, dtype=jnp.int32)
                   % (NPAD - N)).reshape(1, NW, ppt)
    pad_dst = jnp.broadcast_to(pad_dst, (4, NW, ppt))
    srcp = jnp.concatenate(
        [srcs.reshape(4, NW, rpt_e),
         jnp.zeros((4, NW, ppt), jnp.int32)], axis=2).reshape(4, EPAD)
    dstp = jnp.concatenate(
        [dsts.reshape(4, NW, rpt_e), pad_dst], axis=2).reshape(4, EPAD)
    dst2 = dstp.reshape(4, NW, NGE, GE)
    src2d = srcp.reshape(4, NW, NGED, GED)
    dst2d = dstp.reshape(4, NW, NGED, GED)

    zden = jnp.zeros((RPT, 128), jnp.float32)
    zacc = jnp.zeros((RPT, D), jnp.float32)

    hT = _run_proj(x, Wstk)
    elp, erp, mshift = _run_logits(x, Wstk, alstk, arstk)

    wstk, dpstk = _run_att(elp, erp, mshift, srcp, dstp, zden)

    rden = _run_rden(dpstk)
    alphaT = _run_alpha(wstk, rden, dst2)
    part = _run_agg(hT, alphaT, src2d, dst2d)

    return _run_combine(x, part, rWstk, bstk)
